# baseline (speedup 1.0000x reference)
import functools

import jax
import jax.numpy as jnp
from jax import lax
from jax.experimental import pallas as pl
from jax.experimental.pallas import tpu as pltpu
from jax.experimental.pallas import tpu_sc as plsc

H = 128
L = 16
NC = 2
NS = 16
NW = NC * NS
B, S = 4, 4096
TOKENS = B * S
TOK_PER_W = TOKENS // NW
CHUNK = 128
NCHUNK = TOK_PER_W // CHUNK
EPS = 1e-6


def _rsqrt(x):
    i = lax.bitcast_convert_type(x, jnp.int32)
    i = jnp.int32(0x5F3759DF) - lax.shift_right_logical(i, 1)
    y = lax.bitcast_convert_type(i, jnp.float32)
    y = y * (1.5 - 0.5 * x * y * y)
    y = y * (1.5 - 0.5 * x * y * y)
    return y


def _emb_ln_body(wid_hbm, pid_hbm, tid_hbm, wtab_hbm, ptab_hbm, ttab_hbm,
                 gamma_hbm, beta_hbm, out_hbm,
                 idxw_v, idxp_v, idxt_v, rows_w, rows_p, typ_v, gb_v,
                 semw, semp):
    w = lax.axis_index("s") * NC + lax.axis_index("c")
    pltpu.sync_copy(gamma_hbm, gb_v.at[0])
    pltpu.sync_copy(beta_hbm, gb_v.at[1])
    pltpu.sync_copy(ttab_hbm, typ_v)

    for c in range(NCHUNK):
        base = w * TOK_PER_W + c * CHUNK
        pltpu.sync_copy(wid_hbm.at[pl.ds(base, CHUNK)], idxw_v)
        pltpu.sync_copy(pid_hbm.at[pl.ds(base, CHUNK)], idxp_v)
        pltpu.sync_copy(tid_hbm.at[pl.ds(base, CHUNK)], idxt_v)
        cw = pltpu.async_copy(wtab_hbm.at[idxw_v], rows_w, semw)
        cp = pltpu.async_copy(ptab_hbm.at[idxp_v], rows_p, semp)
        cw.wait()
        cp.wait()

        def gbody(g, carry):
            tvec = idxt_v[pl.ds(g * L, L)].astype(jnp.float32)
            for k in range(L):
                t = g * L + k
                tidf = tvec[k]
                xs = []
                for j in range(H // L):
                    sl = pl.ds(j * L, L)
                    t0 = typ_v[0, sl]
                    t1 = typ_v[1, sl]
                    xs.append(rows_w[t, sl] + rows_p[t, sl]
                              + (t0 + tidf * (t1 - t0)))
                s1 = xs
                s2 = [x * x for x in xs]
                while len(s1) > 1:
                    s1 = [s1[i] + s1[i + 1] for i in range(0, len(s1), 2)]
                    s2 = [s2[i] + s2[i + 1] for i in range(0, len(s2), 2)]
                tot1 = jnp.sum(s1[0])
                tot2 = jnp.sum(s2[0])
                mean = tot1 * (1.0 / H)
                var = tot2 * (1.0 / H) - mean * mean
                inv = _rsqrt(var + EPS)
                shift = -mean * inv
                for j in range(H // L):
                    sl = pl.ds(j * L, L)
                    y = (xs[j] * inv + shift) * gb_v[0, sl] + gb_v[1, sl]
                    rows_w[t, sl] = y
            return carry

        lax.fori_loop(0, CHUNK // L, gbody, 0)
        pltpu.sync_copy(rows_w, out_hbm.at[pl.ds(base, CHUNK)])


@functools.partial(jax.jit, static_argnums=())
def _run(word_ids, pos_ids, type_ids, word_table, pos_table, type_table,
         ln_gamma, ln_beta):
    mesh = plsc.VectorSubcoreMesh(core_axis_name="c", subcore_axis_name="s")
    k = pl.kernel(
        _emb_ln_body,
        mesh=mesh,
        compiler_params=pltpu.CompilerParams(needs_layout_passes=False),
        out_type=jax.ShapeDtypeStruct((TOKENS, H), jnp.float32),
        scratch_types=[
            pltpu.VMEM((CHUNK,), jnp.int32),
            pltpu.VMEM((CHUNK,), jnp.int32),
            pltpu.VMEM((CHUNK,), jnp.int32),
            pltpu.VMEM((CHUNK, H), jnp.float32),
            pltpu.VMEM((CHUNK, H), jnp.float32),
            pltpu.VMEM((2, H), jnp.float32),
            pltpu.VMEM((2, H), jnp.float32),
            pltpu.SemaphoreType.DMA,
            pltpu.SemaphoreType.DMA,
        ],
    )
    out = k(word_ids.reshape(TOKENS), pos_ids.reshape(TOKENS),
            type_ids.reshape(TOKENS), word_table, pos_table, type_table,
            ln_gamma, ln_beta)
    return out.reshape(B, S, H)


def kernel(word_ids, pos_ids, type_ids, word_table, pos_table, type_table,
           ln_gamma, ln_beta):
    return _run(word_ids, pos_ids, type_ids, word_table, pos_table,
                type_table, ln_gamma, ln_beta)


# speedup vs baseline: 2.2477x; 2.2477x over previous
import functools

import jax
import jax.numpy as jnp
from jax import lax
from jax.experimental import pallas as pl
from jax.experimental.pallas import tpu as pltpu
from jax.experimental.pallas import tpu_sc as plsc

H = 128
L = 16
NC = 2
NS = 16
NW = NC * NS
B, S = 4, 4096
TOKENS = B * S
TOK_PER_W = TOKENS // NW
CHUNK = 128
NCHUNK = TOK_PER_W // CHUNK
EPS = 1e-6

TC_ROWS = 2048


def _gather_sum_body(wid_hbm, pid_hbm, wtab_hbm, ptab_hbm, out_hbm,
                     idxw_v, idxp_v, rows_w, rows_p, semw, semp):
    w = lax.axis_index("s") * NC + lax.axis_index("c")

    for c in range(NCHUNK):
        base = w * TOK_PER_W + c * CHUNK
        pltpu.sync_copy(wid_hbm.at[pl.ds(base, CHUNK)], idxw_v)
        pltpu.sync_copy(pid_hbm.at[pl.ds(base, CHUNK)], idxp_v)
        cw = pltpu.async_copy(wtab_hbm.at[idxw_v], rows_w, semw)
        cp = pltpu.async_copy(ptab_hbm.at[idxp_v], rows_p, semp)
        cw.wait()
        cp.wait()

        def body(t, carry):
            for j in range(H // L):
                sl = pl.ds(j * L, L)
                rows_w[t, sl] = rows_w[t, sl] + rows_p[t, sl]
            return carry

        lax.fori_loop(0, CHUNK, body, 0, unroll=4)
        pltpu.sync_copy(rows_w, out_hbm.at[pl.ds(base, CHUNK)])


def _ln_tc_kernel(sum_ref, tidf_ref, ttab_ref, gamma_ref, beta_ref, out_ref):
    x = sum_ref[...]
    tidf = tidf_ref[...]
    t0 = ttab_ref[0:1, :]
    t1 = ttab_ref[1:2, :]
    x = x + t0 + tidf * (t1 - t0)
    mean = jnp.mean(x, axis=-1, keepdims=True)
    var = jnp.mean(jnp.square(x - mean), axis=-1, keepdims=True)
    normed = (x - mean) * lax.rsqrt(var + EPS)
    out_ref[...] = normed * gamma_ref[0, :] + beta_ref[0, :]


@jax.jit
def _run(word_ids, pos_ids, type_ids, word_table, pos_table, type_table,
         ln_gamma, ln_beta):
    mesh = plsc.VectorSubcoreMesh(core_axis_name="c", subcore_axis_name="s")
    sc_k = pl.kernel(
        _gather_sum_body,
        mesh=mesh,
        compiler_params=pltpu.CompilerParams(needs_layout_passes=False),
        out_type=jax.ShapeDtypeStruct((TOKENS, H), jnp.float32),
        scratch_types=[
            pltpu.VMEM((CHUNK,), jnp.int32),
            pltpu.VMEM((CHUNK,), jnp.int32),
            pltpu.VMEM((CHUNK, H), jnp.float32),
            pltpu.VMEM((CHUNK, H), jnp.float32),
            pltpu.SemaphoreType.DMA,
            pltpu.SemaphoreType.DMA,
        ],
    )
    summed = sc_k(word_ids.reshape(TOKENS), pos_ids.reshape(TOKENS),
                  word_table, pos_table)

    ngrid = TOKENS // TC_ROWS
    tidf = type_ids.reshape(TOKENS, 1).astype(jnp.float32)
    out = pl.pallas_call(
        _ln_tc_kernel,
        grid=(ngrid,),
        in_specs=[
            pl.BlockSpec((TC_ROWS, H), lambda i: (i, 0)),
            pl.BlockSpec((TC_ROWS, 1), lambda i: (i, 0)),
            pl.BlockSpec((2, H), lambda i: (0, 0)),
            pl.BlockSpec((1, H), lambda i: (0, 0)),
            pl.BlockSpec((1, H), lambda i: (0, 0)),
        ],
        out_specs=pl.BlockSpec((TC_ROWS, H), lambda i: (i, 0)),
        out_shape=jax.ShapeDtypeStruct((TOKENS, H), jnp.float32),
    )(summed, tidf, type_table, ln_gamma.reshape(1, H),
      ln_beta.reshape(1, H))
    return out.reshape(B, S, H)


def kernel(word_ids, pos_ids, type_ids, word_table, pos_table, type_table,
           ln_gamma, ln_beta):
    return _run(word_ids, pos_ids, type_ids, word_table, pos_table,
                type_table, ln_gamma, ln_beta)


# speedup vs baseline: 2.9113x; 1.2952x over previous
import functools

import jax
import jax.numpy as jnp
from jax import lax
from jax.experimental import pallas as pl
from jax.experimental.pallas import tpu as pltpu
from jax.experimental.pallas import tpu_sc as plsc

H = 128
L = 16
NC = 2
NS = 16
NW = NC * NS
B, S = 4, 4096
TOKENS = B * S
TOK_PER_W = TOKENS // NW
CHUNK = 128
NCHUNK = TOK_PER_W // CHUNK
EPS = 1e-6

TC_ROWS = 2048


def _gather_sum_body(wid_hbm, pid_hbm, wtab_hbm, ptab_hbm, out_hbm,
                     idxw_v, idxp_v, rows_w, rows_p, semw, semp):
    w = lax.axis_index("s") * NC + lax.axis_index("c")
    base_w = w * TOK_PER_W

    pltpu.sync_copy(wid_hbm.at[pl.ds(base_w, TOK_PER_W)], idxw_v)
    pltpu.sync_copy(pid_hbm.at[pl.ds(base_w, TOK_PER_W)], idxp_v)

    def start(c):
        b = c % 2
        cw = pltpu.async_copy(
            wtab_hbm.at[idxw_v.at[pl.ds(c * CHUNK, CHUNK)]], rows_w.at[b],
            semw)
        cp = pltpu.async_copy(
            ptab_hbm.at[idxp_v.at[pl.ds(c * CHUNK, CHUNK)]], rows_p.at[b],
            semp)
        return cw, cp

    pend = start(0)
    for c in range(NCHUNK):
        b = c % 2
        cw, cp = pend
        cw.wait()
        cp.wait()
        if c + 1 < NCHUNK:
            pend = start(c + 1)

        def body(t, carry):
            for j in range(H // L):
                sl = pl.ds(j * L, L)
                rows_w[b, t, sl] = rows_w[b, t, sl] + rows_p[b, t, sl]
            return carry

        lax.fori_loop(0, CHUNK, body, 0, unroll=4)
        pltpu.sync_copy(rows_w.at[b],
                        out_hbm.at[pl.ds(base_w + c * CHUNK, CHUNK)])


def _ln_tc_kernel(sum_ref, tidf_ref, ttab_ref, gamma_ref, beta_ref, out_ref):
    x = sum_ref[...]
    tidf = tidf_ref[...]
    t0 = ttab_ref[0:1, :]
    t1 = ttab_ref[1:2, :]
    x = x + t0 + tidf * (t1 - t0)
    mean = jnp.mean(x, axis=-1, keepdims=True)
    var = jnp.mean(jnp.square(x - mean), axis=-1, keepdims=True)
    normed = (x - mean) * lax.rsqrt(var + EPS)
    out_ref[...] = normed * gamma_ref[0, :] + beta_ref[0, :]


@jax.jit
def _run(word_ids, pos_ids, type_ids, word_table, pos_table, type_table,
         ln_gamma, ln_beta):
    mesh = plsc.VectorSubcoreMesh(core_axis_name="c", subcore_axis_name="s")
    sc_k = pl.kernel(
        _gather_sum_body,
        mesh=mesh,
        compiler_params=pltpu.CompilerParams(needs_layout_passes=False),
        out_type=jax.ShapeDtypeStruct((TOKENS, H), jnp.float32),
        scratch_types=[
            pltpu.VMEM((TOK_PER_W,), jnp.int32),
            pltpu.VMEM((TOK_PER_W,), jnp.int32),
            pltpu.VMEM((2, CHUNK, H), jnp.float32),
            pltpu.VMEM((2, CHUNK, H), jnp.float32),
            pltpu.SemaphoreType.DMA,
            pltpu.SemaphoreType.DMA,
        ],
    )
    summed = sc_k(word_ids.reshape(TOKENS), pos_ids.reshape(TOKENS),
                  word_table, pos_table)

    ngrid = TOKENS // TC_ROWS
    tidf = type_ids.reshape(TOKENS, 1).astype(jnp.float32)
    out = pl.pallas_call(
        _ln_tc_kernel,
        grid=(ngrid,),
        in_specs=[
            pl.BlockSpec((TC_ROWS, H), lambda i: (i, 0)),
            pl.BlockSpec((TC_ROWS, 1), lambda i: (i, 0)),
            pl.BlockSpec((2, H), lambda i: (0, 0)),
            pl.BlockSpec((1, H), lambda i: (0, 0)),
            pl.BlockSpec((1, H), lambda i: (0, 0)),
        ],
        out_specs=pl.BlockSpec((TC_ROWS, H), lambda i: (i, 0)),
        out_shape=jax.ShapeDtypeStruct((TOKENS, H), jnp.float32),
    )(summed, tidf, type_table, ln_gamma.reshape(1, H),
      ln_beta.reshape(1, H))
    return out.reshape(B, S, H)


def kernel(word_ids, pos_ids, type_ids, word_table, pos_table, type_table,
           ln_gamma, ln_beta):
    return _run(word_ids, pos_ids, type_ids, word_table, pos_table,
                type_table, ln_gamma, ln_beta)


# speedup vs baseline: 2.9128x; 1.0005x over previous
import functools

import jax
import jax.numpy as jnp
from jax import lax
from jax.experimental import pallas as pl
from jax.experimental.pallas import tpu as pltpu
from jax.experimental.pallas import tpu_sc as plsc

H = 128
L = 16
NC = 2
NS = 16
NW = NC * NS
B, S = 4, 4096
TOKENS = B * S
TOK_PER_W = TOKENS // NW
CHUNK = 128
NCHUNK = TOK_PER_W // CHUNK
EPS = 1e-6

TC_ROWS = 2048


def _gather_sum_body(wid_hbm, pid_hbm, wtab_hbm, ptab_hbm, out_hbm,
                     idxw_v, idxp_v, rows_w, rows_p, semw, semp):
    w = lax.axis_index("s") * NC + lax.axis_index("c")
    base_w = w * TOK_PER_W

    pltpu.sync_copy(wid_hbm.at[pl.ds(base_w, TOK_PER_W)], idxw_v)
    pltpu.sync_copy(pid_hbm.at[pl.ds(base_w, TOK_PER_W)], idxp_v)

    def start(c):
        b = c % 2
        cw = pltpu.async_copy(
            wtab_hbm.at[idxw_v.at[pl.ds(c * CHUNK, CHUNK)]], rows_w.at[b],
            semw)
        cp = pltpu.async_copy(
            ptab_hbm.at[idxp_v.at[pl.ds(c * CHUNK, CHUNK)]], rows_p.at[b],
            semp)
        return cw, cp

    pend = start(0)
    for c in range(NCHUNK):
        b = c % 2
        cw, cp = pend
        cw.wait()
        cp.wait()
        if c + 1 < NCHUNK:
            pend = start(c + 1)

        def body(t, carry):
            for j in range(H // L):
                sl = pl.ds(j * L, L)
                rows_w[b, t, sl] = rows_w[b, t, sl] + rows_p[b, t, sl]
            return carry

        lax.fori_loop(0, CHUNK, body, 0, unroll=4)
        pltpu.sync_copy(rows_w.at[b],
                        out_hbm.at[pl.ds(base_w + c * CHUNK, CHUNK)])


def _ln_tc_kernel(sum_ref, tid_ref, ttab_ref, gamma_ref, beta_ref, out_ref):
    x = sum_ref[...]
    tidf = tid_ref[...].astype(jnp.float32)
    t0 = ttab_ref[0:1, :]
    t1 = ttab_ref[1:2, :]
    x = x + t0 + tidf * (t1 - t0)
    mean = jnp.mean(x, axis=-1, keepdims=True)
    var = jnp.mean(jnp.square(x - mean), axis=-1, keepdims=True)
    normed = (x - mean) * lax.rsqrt(var + EPS)
    out_ref[...] = normed * gamma_ref[0, :] + beta_ref[0, :]


@jax.jit
def _run(word_ids, pos_ids, type_ids, word_table, pos_table, type_table,
         ln_gamma, ln_beta):
    mesh = plsc.VectorSubcoreMesh(core_axis_name="c", subcore_axis_name="s")
    sc_k = pl.kernel(
        _gather_sum_body,
        mesh=mesh,
        compiler_params=pltpu.CompilerParams(needs_layout_passes=False),
        out_type=jax.ShapeDtypeStruct((TOKENS, H), jnp.float32),
        scratch_types=[
            pltpu.VMEM((TOK_PER_W,), jnp.int32),
            pltpu.VMEM((TOK_PER_W,), jnp.int32),
            pltpu.VMEM((2, CHUNK, H), jnp.float32),
            pltpu.VMEM((2, CHUNK, H), jnp.float32),
            pltpu.SemaphoreType.DMA,
            pltpu.SemaphoreType.DMA,
        ],
    )
    summed = sc_k(word_ids.reshape(TOKENS), pos_ids.reshape(TOKENS),
                  word_table, pos_table)

    ngrid = TOKENS // TC_ROWS
    tids = type_ids.reshape(TOKENS, 1)
    out = pl.pallas_call(
        _ln_tc_kernel,
        grid=(ngrid,),
        in_specs=[
            pl.BlockSpec((TC_ROWS, H), lambda i: (i, 0)),
            pl.BlockSpec((TC_ROWS, 1), lambda i: (i, 0)),
            pl.BlockSpec((2, H), lambda i: (0, 0)),
            pl.BlockSpec((1, H), lambda i: (0, 0)),
            pl.BlockSpec((1, H), lambda i: (0, 0)),
        ],
        out_specs=pl.BlockSpec((TC_ROWS, H), lambda i: (i, 0)),
        out_shape=jax.ShapeDtypeStruct((TOKENS, H), jnp.float32),
    )(summed, tids, type_table, ln_gamma.reshape(1, H),
      ln_beta.reshape(1, H))
    return out.reshape(B, S, H)


def kernel(word_ids, pos_ids, type_ids, word_table, pos_table, type_table,
           ln_gamma, ln_beta):
    return _run(word_ids, pos_ids, type_ids, word_table, pos_table,
                type_table, ln_gamma, ln_beta)
